# baseline (speedup 1.0000x reference)
import jax
import jax.numpy as jnp
from jax.experimental import pallas as pl
from jax.experimental.pallas import tpu as pltpu

_FANO_LINES = [(0, 1, 2), (0, 3, 4), (0, 5, 6), (1, 3, 5), (1, 4, 6), (2, 3, 6), (2, 4, 5)]
_LINES_OF = [[0, 1, 2], [0, 3, 4], [0, 5, 6], [1, 3, 5], [1, 4, 6], [2, 3, 6], [2, 4, 5]]
_NUM_LAYERS = 3
_VQ_LEVELS = 8
_CDIM = 8
_TDIM = 7 * _CDIM
_K = 240

_HI = jax.lax.Precision.HIGHEST


def _gram_body(cb_ref, cbt_ref, A1_ref, A2_ref, A3_ref):
    f32 = jnp.float32
    mask = jnp.uint32(0xFFFF0000)
    G2 = 2.0 * jnp.dot(cb_ref[...], cbt_ref[...], preferred_element_type=f32,
                       precision=_HI)
    a1 = jax.lax.bitcast_convert_type(
        jax.lax.bitcast_convert_type(G2, jnp.uint32) & mask, f32)
    rem = G2 - a1
    a2 = jax.lax.bitcast_convert_type(
        jax.lax.bitcast_convert_type(rem, jnp.uint32) & mask, f32)
    A1_ref[...] = a1
    A2_ref[...] = a2
    A3_ref[...] = rem - a2


def _main_body(x_ref, Wenc_ref, MT_ref, bcatT_ref, wrepT_ref, Wto8T_ref,
               cb_ref, c1t_ref, c2t_ref, c3t_ref, A1_ref, A2_ref, A3_ref,
               Wfrom8_ref, Wdec_ref, out_ref):
    f32 = jnp.float32
    B = x_ref.shape[0]
    C = cb_ref[...]
    cn = jnp.sum(C * C, axis=1, keepdims=True)
    iota = jax.lax.broadcasted_iota(jnp.int32, (_K, B), 0).astype(f32)

    h = jnp.dot(x_ref[...], Wenc_ref[...], preferred_element_type=f32)

    hT = h.T
    for l in range(_NUM_LAYERS):
        outsT = jnp.dot(MT_ref[l], hT, preferred_element_type=f32)
        outsT = (outsT + bcatT_ref[:, l:l + 1]) * wrepT_ref[:, l:l + 1]
        res = []
        for c in range(7):
            a, b, g = _LINES_OF[c]
            res.append(outsT[a * _CDIM:(a + 1) * _CDIM, :]
                       + outsT[b * _CDIM:(b + 1) * _CDIM, :]
                       + outsT[g * _CDIM:(g + 1) * _CDIM, :])
        hT = jnp.concatenate(res, axis=0) / 3.0 + hT
    zT = jnp.dot(Wto8T_ref[...], hT, preferred_element_type=f32)

    sT = jnp.dot(C, zT, preferred_element_type=f32, precision=_HI)
    d = cn - 2.0 * sT
    OH = jnp.zeros((_K, B), f32)
    for lvl in range(_VQ_LEVELS):
        m = jnp.min(d, axis=0, keepdims=True)
        idx = jnp.min(jnp.where(d == m, iota, float(_K)), axis=0,
                      keepdims=True)
        oh = (iota == idx).astype(f32)
        OH = OH + oh
        if lvl + 1 < _VQ_LEVELS:
            g2 = ((jnp.dot(A1_ref[...], oh, preferred_element_type=f32)
                   + jnp.dot(A2_ref[...], oh, preferred_element_type=f32))
                  + jnp.dot(A3_ref[...], oh, preferred_element_type=f32))
            d = d + g2
    qT = (jnp.dot(c1t_ref[...], OH, preferred_element_type=f32)
          + jnp.dot(c2t_ref[...], OH, preferred_element_type=f32)
          + jnp.dot(c3t_ref[...], OH, preferred_element_type=f32))
    zqT = zT + (qT - zT)
    zq = zqT.T
    t = jnp.dot(zq, Wfrom8_ref[...], preferred_element_type=f32)
    out_ref[...] = jnp.dot(t, Wdec_ref[...], preferred_element_type=f32)


def kernel(x, W_enc, fano_W, fano_b, line_weights, W_to8, codebook, W_from8, W_dec):
    n, bulk = x.shape
    f32 = jnp.float32

    ws = jnp.stack([jax.nn.softmax(line_weights[l]) for l in range(_NUM_LAYERS)])
    cols = []
    for li, (i, j, k) in enumerate(_FANO_LINES):
        blk = jnp.zeros((_NUM_LAYERS, _TDIM, _CDIM), f32)
        blk = blk.at[:, _CDIM * i:_CDIM * (i + 1), :].set(fano_W[:, li, 0:8])
        blk = blk.at[:, _CDIM * j:_CDIM * (j + 1), :].set(fano_W[:, li, 8:16])
        blk = blk.at[:, _CDIM * k:_CDIM * (k + 1), :].set(fano_W[:, li, 16:24])
        cols.append(blk)
    M = jnp.concatenate(cols, axis=2)
    MT = jnp.transpose(M, (0, 2, 1))
    b_catT = fano_b.reshape(_NUM_LAYERS, _TDIM).T
    w_repT = jnp.repeat(ws, _CDIM, axis=1).T

    mask = jnp.uint32(0xFFFF0000)
    bits = jax.lax.bitcast_convert_type(codebook, jnp.uint32)
    c1 = jax.lax.bitcast_convert_type(bits & mask, f32)
    r1 = codebook - c1
    c2 = jax.lax.bitcast_convert_type(
        jax.lax.bitcast_convert_type(r1, jnp.uint32) & mask, f32)
    c3 = r1 - c2

    A1, A2, A3 = pl.pallas_call(
        _gram_body,
        out_shape=[jax.ShapeDtypeStruct((_K, _K), f32)] * 3,
    )(codebook, codebook.T)

    blk = 4096
    grid = (n // blk,)
    out = pl.pallas_call(
        _main_body,
        grid=grid,
        in_specs=[
            pl.BlockSpec((blk, bulk), lambda i: (i, 0)),
            pl.BlockSpec((bulk, _TDIM), lambda i: (0, 0)),
            pl.BlockSpec((_NUM_LAYERS, _TDIM, _TDIM), lambda i: (0, 0, 0)),
            pl.BlockSpec((_TDIM, _NUM_LAYERS), lambda i: (0, 0)),
            pl.BlockSpec((_TDIM, _NUM_LAYERS), lambda i: (0, 0)),
            pl.BlockSpec((_CDIM, _TDIM), lambda i: (0, 0)),
            pl.BlockSpec((_K, _CDIM), lambda i: (0, 0)),
            pl.BlockSpec((_CDIM, _K), lambda i: (0, 0)),
            pl.BlockSpec((_CDIM, _K), lambda i: (0, 0)),
            pl.BlockSpec((_CDIM, _K), lambda i: (0, 0)),
            pl.BlockSpec((_K, _K), lambda i: (0, 0)),
            pl.BlockSpec((_K, _K), lambda i: (0, 0)),
            pl.BlockSpec((_K, _K), lambda i: (0, 0)),
            pl.BlockSpec((_CDIM, _TDIM), lambda i: (0, 0)),
            pl.BlockSpec((_TDIM, bulk), lambda i: (0, 0)),
        ],
        out_specs=pl.BlockSpec((blk, bulk), lambda i: (i, 0)),
        out_shape=jax.ShapeDtypeStruct((n, bulk), f32),
        compiler_params=pltpu.CompilerParams(
            dimension_semantics=("arbitrary",),
        ),
    )(x, W_enc, MT, b_catT, w_repT, W_to8.T, codebook, c1.T, c2.T, c3.T,
      A1, A2, A3, W_from8, W_dec)
    return out


# speedup vs baseline: 1.0737x; 1.0737x over previous
import jax
import jax.numpy as jnp
from jax.experimental import pallas as pl
from jax.experimental.pallas import tpu as pltpu

_FANO_LINES = [(0, 1, 2), (0, 3, 4), (0, 5, 6), (1, 3, 5), (1, 4, 6), (2, 3, 6), (2, 4, 5)]
_LINES_OF = [[0, 1, 2], [0, 3, 4], [0, 5, 6], [1, 3, 5], [1, 4, 6], [2, 3, 6], [2, 4, 5]]
_NUM_LAYERS = 3
_VQ_LEVELS = 8
_CDIM = 8
_TDIM = 7 * _CDIM
_K = 240

_HI = jax.lax.Precision.HIGHEST


def _gram_body(cb_ref, cbt_ref, A1_ref, A2_ref, A3_ref):
    f32 = jnp.float32
    mask = jnp.uint32(0xFFFF0000)
    G2 = 2.0 * jnp.dot(cb_ref[...], cbt_ref[...], preferred_element_type=f32,
                       precision=_HI)
    a1 = jax.lax.bitcast_convert_type(
        jax.lax.bitcast_convert_type(G2, jnp.uint32) & mask, f32)
    rem = G2 - a1
    a2 = jax.lax.bitcast_convert_type(
        jax.lax.bitcast_convert_type(rem, jnp.uint32) & mask, f32)
    A1_ref[...] = a1
    A2_ref[...] = a2
    A3_ref[...] = rem - a2


def _main_body(x_ref, Wenc_ref, MT_ref, bcatT_ref, wrepT_ref, Wto8T_ref,
               cb_ref, c1t_ref, c2t_ref, c3t_ref, A1_ref, A2_ref, A3_ref,
               Wfrom8_ref, Wdec_ref, out_ref):
    f32 = jnp.float32
    B = x_ref.shape[0]
    C = cb_ref[...]
    cn = jnp.sum(C * C, axis=1, keepdims=True)
    iota = jax.lax.broadcasted_iota(jnp.int32, (_K, B), 0).astype(f32)

    h = jnp.dot(x_ref[...], Wenc_ref[...], preferred_element_type=f32)

    hT = h.T
    for l in range(_NUM_LAYERS):
        outsT = jnp.dot(MT_ref[l], hT, preferred_element_type=f32)
        outsT = (outsT + bcatT_ref[:, l:l + 1]) * wrepT_ref[:, l:l + 1]
        res = []
        for c in range(7):
            a, b, g = _LINES_OF[c]
            res.append(outsT[a * _CDIM:(a + 1) * _CDIM, :]
                       + outsT[b * _CDIM:(b + 1) * _CDIM, :]
                       + outsT[g * _CDIM:(g + 1) * _CDIM, :])
        hT = jnp.concatenate(res, axis=0) / 3.0 + hT
    zT = jnp.dot(Wto8T_ref[...], hT, preferred_element_type=f32)

    sT = jnp.dot(C, zT, preferred_element_type=f32, precision=_HI)
    d = cn - 2.0 * sT
    OH = jnp.zeros((_K, B), f32)
    for lvl in range(_VQ_LEVELS):
        m = jnp.min(d, axis=0, keepdims=True)
        idx = jnp.min(jnp.where(d == m, iota, float(_K)), axis=0,
                      keepdims=True)
        oh = (iota == idx).astype(f32)
        OH = OH + oh
        if lvl + 1 < _VQ_LEVELS:
            g2 = ((jnp.dot(A1_ref[...], oh, preferred_element_type=f32)
                   + jnp.dot(A2_ref[...], oh, preferred_element_type=f32))
                  + jnp.dot(A3_ref[...], oh, preferred_element_type=f32))
            d = d + g2
    qT = (jnp.dot(c1t_ref[...], OH, preferred_element_type=f32)
          + jnp.dot(c2t_ref[...], OH, preferred_element_type=f32)
          + jnp.dot(c3t_ref[...], OH, preferred_element_type=f32))
    zqT = zT + (qT - zT)
    zq = zqT.T
    t = jnp.dot(zq, Wfrom8_ref[...], preferred_element_type=f32)
    out_ref[...] = jnp.dot(t, Wdec_ref[...], preferred_element_type=f32)


def kernel(x, W_enc, fano_W, fano_b, line_weights, W_to8, codebook, W_from8, W_dec):
    n, bulk = x.shape
    f32 = jnp.float32

    ws = jnp.stack([jax.nn.softmax(line_weights[l]) for l in range(_NUM_LAYERS)])
    cols = []
    for li, (i, j, k) in enumerate(_FANO_LINES):
        blk = jnp.zeros((_NUM_LAYERS, _TDIM, _CDIM), f32)
        blk = blk.at[:, _CDIM * i:_CDIM * (i + 1), :].set(fano_W[:, li, 0:8])
        blk = blk.at[:, _CDIM * j:_CDIM * (j + 1), :].set(fano_W[:, li, 8:16])
        blk = blk.at[:, _CDIM * k:_CDIM * (k + 1), :].set(fano_W[:, li, 16:24])
        cols.append(blk)
    M = jnp.concatenate(cols, axis=2)
    MT = jnp.transpose(M, (0, 2, 1))
    b_catT = fano_b.reshape(_NUM_LAYERS, _TDIM).T
    w_repT = jnp.repeat(ws, _CDIM, axis=1).T

    mask = jnp.uint32(0xFFFF0000)
    bits = jax.lax.bitcast_convert_type(codebook, jnp.uint32)
    c1 = jax.lax.bitcast_convert_type(bits & mask, f32)
    r1 = codebook - c1
    c2 = jax.lax.bitcast_convert_type(
        jax.lax.bitcast_convert_type(r1, jnp.uint32) & mask, f32)
    c3 = r1 - c2

    A1, A2, A3 = pl.pallas_call(
        _gram_body,
        out_shape=[jax.ShapeDtypeStruct((_K, _K), f32)] * 3,
    )(codebook, codebook.T)

    blk = 2048
    grid = (n // blk,)
    out = pl.pallas_call(
        _main_body,
        grid=grid,
        in_specs=[
            pl.BlockSpec((blk, bulk), lambda i: (i, 0)),
            pl.BlockSpec((bulk, _TDIM), lambda i: (0, 0)),
            pl.BlockSpec((_NUM_LAYERS, _TDIM, _TDIM), lambda i: (0, 0, 0)),
            pl.BlockSpec((_TDIM, _NUM_LAYERS), lambda i: (0, 0)),
            pl.BlockSpec((_TDIM, _NUM_LAYERS), lambda i: (0, 0)),
            pl.BlockSpec((_CDIM, _TDIM), lambda i: (0, 0)),
            pl.BlockSpec((_K, _CDIM), lambda i: (0, 0)),
            pl.BlockSpec((_CDIM, _K), lambda i: (0, 0)),
            pl.BlockSpec((_CDIM, _K), lambda i: (0, 0)),
            pl.BlockSpec((_CDIM, _K), lambda i: (0, 0)),
            pl.BlockSpec((_K, _K), lambda i: (0, 0)),
            pl.BlockSpec((_K, _K), lambda i: (0, 0)),
            pl.BlockSpec((_K, _K), lambda i: (0, 0)),
            pl.BlockSpec((_CDIM, _TDIM), lambda i: (0, 0)),
            pl.BlockSpec((_TDIM, bulk), lambda i: (0, 0)),
        ],
        out_specs=pl.BlockSpec((blk, bulk), lambda i: (i, 0)),
        out_shape=jax.ShapeDtypeStruct((n, bulk), f32),
        compiler_params=pltpu.CompilerParams(
            dimension_semantics=("arbitrary",),
        ),
    )(x, W_enc, MT, b_catT, w_repT, W_to8.T, codebook, c1.T, c2.T, c3.T,
      A1, A2, A3, W_from8, W_dec)
    return out
